# Initial kernel scaffold; baseline (speedup 1.0000x reference)
import jax
import jax.numpy as jnp
from jax.experimental import pallas as pl


def kernel(x, adj, W):
    raise NotImplementedError("write your pallas kernel here")


# speedup vs baseline: 1.8523x; 1.8523x over previous
import jax
import jax.numpy as jnp
from jax.experimental import pallas as pl

_BM = 400


def _sage_body(adj_ref, xb_ref, xi_ref, w1_ref, w2_ref, out_ref):
    a = adj_ref[...]
    acc = jnp.dot(a.astype(jnp.bfloat16), xb_ref[...],
                  preferred_element_type=jnp.float32)
    s = jnp.sum(a, axis=1, keepdims=True)
    neigh = acc / (s + 1.0)
    out_ref[...] = jnp.maximum(
        jnp.dot(xi_ref[...], w1_ref[...], preferred_element_type=jnp.float32)
        + jnp.dot(neigh, w2_ref[...], preferred_element_type=jnp.float32),
        0.0,
    )


def kernel(x, adj, W):
    n, nfeat = x.shape
    nhid = W.shape[0]
    xb = x.astype(jnp.bfloat16)
    w1 = W[:, :nfeat].T
    w2 = W[:, nfeat:].T
    return pl.pallas_call(
        _sage_body,
        grid=(n // _BM,),
        in_specs=[
            pl.BlockSpec((_BM, n), lambda i: (i, 0)),
            pl.BlockSpec((n, nfeat), lambda i: (0, 0)),
            pl.BlockSpec((_BM, nfeat), lambda i: (i, 0)),
            pl.BlockSpec((nfeat, nhid), lambda i: (0, 0)),
            pl.BlockSpec((nfeat, nhid), lambda i: (0, 0)),
        ],
        out_specs=pl.BlockSpec((_BM, nhid), lambda i: (i, 0)),
        out_shape=jax.ShapeDtypeStruct((n, nhid), x.dtype),
    )(adj, xb, x, w1, w2)
